# baseline (speedup 1.0000x reference)
import functools

import jax
import jax.numpy as jnp
from jax import lax
from jax.experimental import pallas as pl
from jax.experimental.pallas import tpu as pltpu
from jax.experimental.pallas import tpu_sc as plsc

_NC = 2
_NS = 16
_NW = _NC * _NS
_C = 128
_G = 1
_NBUF = 6


@functools.partial(jax.jit, static_argnames=("cpw", "dim"))
def _gather_sc(idx3, table, cpw, dim):
    n = _NW * cpw * _C
    m = cpw
    assert m > _NBUF
    mesh = plsc.VectorSubcoreMesh(core_axis_name="c", subcore_axis_name="s")

    @functools.partial(
        pl.kernel,
        out_type=jax.ShapeDtypeStruct((n // _C, _C, dim), table.dtype),
        mesh=mesh,
        scratch_types=[
            pltpu.VMEM((cpw, _C), jnp.int32),
            pltpu.VMEM((_NBUF, _C, dim), table.dtype),
            pltpu.SemaphoreType.DMA((_NBUF,)),
            pltpu.SemaphoreType.DMA((_NBUF,)),
        ],
    )
    def k(idx_hbm, table_hbm, out_hbm, idx_v, rows_v, gsem, wsem):
        wid = lax.axis_index("s") * _NC + lax.axis_index("c")
        pltpu.sync_copy(idx_hbm.at[wid], idx_v)
        base = wid * cpw

        def fire_gather(j, b):
            pltpu.async_copy(
                table_hbm.at[idx_v.at[j]], rows_v.at[b], gsem.at[b])

        def wait_gather(j, b):
            pltpu.make_async_copy(
                table_hbm.at[idx_v.at[j]], rows_v.at[b], gsem.at[b]).wait()

        def fire_write(j, b):
            pltpu.async_copy(
                rows_v.at[pl.ds(b, 1)], out_hbm.at[pl.ds(base + j, 1)],
                wsem.at[b])

        def wait_write(j, b):
            pltpu.make_async_copy(
                rows_v.at[pl.ds(b, 1)],
                out_hbm.at[pl.ds(base + j, 1)], wsem.at[b]).wait()

        for b in range(_NBUF):
            fire_gather(b, b)
        for j in range(m):
            b = j % _NBUF
            wait_gather(j, b)
            fire_write(j, b)
            if j + _NBUF < m:
                wait_write(j, b)
                fire_gather(j + _NBUF, b)
        for j in range(m - _NBUF, m):
            wait_write(j, j % _NBUF)

    return k(idx3, table)


def kernel(indices, table):
    b, l = indices.shape
    dim = table.shape[1]
    n = b * l
    assert n % (_NW * _C) == 0
    cpw = n // (_NW * _C)
    idx3 = indices.T.reshape(_NW, cpw, _C)
    out = _gather_sc(idx3, table, cpw, dim)
    return out.reshape(l, b, dim).transpose(1, 0, 2)
